# baseline (speedup 1.0000x reference)
import functools

import jax
import jax.numpy as jnp
from jax import lax
from jax.experimental import pallas as pl
from jax.experimental.pallas import tpu as pltpu
from jax.experimental.pallas import tpu_sc as plsc

B = 2
NVIEW = 5
NJ = 15
H = 128
W = 240
HW = H * W
NB = 64 * 64 * 64
C = 16

NC = 2
NS = 16
NW = NC * NS

P = 256
NCHUNK = NB // NS // P
PPW = NB // NS
NROW = NVIEW * 2
NIR = NROW * (P // 128)
RPC = NROW * P
CP = 2 * C

_mesh = plsc.VectorSubcoreMesh(
    core_axis_name="c", subcore_axis_name="s", num_cores=NC, num_subcores=NS
)


@functools.partial(
    pl.kernel,
    out_type=jax.ShapeDtypeStruct((B * NB * C,), jnp.float32),
    mesh=_mesh,
    scratch_types=[
        pltpu.VMEM((2 * NIR, 128), jnp.int32),
        pltpu.VMEM((2 * RPC, C), jnp.int32),
        pltpu.VMEM((2 * NVIEW * 4 * P,), jnp.float32),
        pltpu.VMEM((2 * NVIEW * P,), jnp.float32),
        pltpu.VMEM((2 * NVIEW * P,), jnp.float32),
        pltpu.VMEM((2 * P * C,), jnp.float32),
        pltpu.SemaphoreType.DMA,
        pltpu.SemaphoreType.DMA,
        pltpu.SemaphoreType.DMA,
        pltpu.SemaphoreType.DMA,
        pltpu.SemaphoreType.DMA,
    ],
    compiler_params=pltpu.CompilerParams(use_tc_tiling_on_sc=False),
)
def _sc_sample(tab, gx, gy, out, idx_ref, rows, wbuf, cbx, cby, outb,
               csem, gsem0, gsem1, osem0, osem1):
    gsem = (gsem0, gsem1)
    osem = (osem0, osem1)
    wid = lax.axis_index("s") * NC + lax.axis_index("c")
    b = wid // NS
    pbase = (wid % NS) * PPW

    def fire_coords(k, par):
        start = pbase + k * P
        for v in range(NVIEW):
            off = (b * NVIEW + v) * NB + start
            pltpu.async_copy(gx.at[pl.ds(off, P)],
                             cbx.at[pl.ds((par * NVIEW + v) * P, P)], csem)
            pltpu.async_copy(gy.at[pl.ds(off, P)],
                             cby.at[pl.ds((par * NVIEW + v) * P, P)], csem)

    def wait_coords(par):
        pltpu.make_async_copy(gx.at[pl.ds(0, NVIEW * P)],
                              cbx.at[pl.ds(par * NVIEW * P, NVIEW * P)],
                              csem).wait()
        pltpu.make_async_copy(gy.at[pl.ds(0, NVIEW * P)],
                              cby.at[pl.ds(par * NVIEW * P, NVIEW * P)],
                              csem).wait()

    def stage_a(par):
        for v in range(NVIEW):
            base_off = (b * NVIEW + v) * HW
            for g in range(P // 16):
                gxv = cbx[pl.ds((par * NVIEW + v) * P + g * 16, 16)]
                gyv = cby[pl.ds((par * NVIEW + v) * P + g * 16, 16)]
                fx = gxv * (0.5 * (W - 1)) + (0.5 * (W - 1))
                fy = gyv * (0.5 * (H - 1)) + (0.5 * (H - 1))
                ix0 = jnp.maximum(jnp.minimum(fx.astype(jnp.int32), W - 2), 0)
                iy0 = jnp.maximum(jnp.minimum(fy.astype(jnp.int32), H - 2), 0)
                wx1 = fx - ix0.astype(jnp.float32)
                wx0 = 1.0 - wx1
                wy1 = fy - iy0.astype(jnp.float32)
                wy0 = 1.0 - wy1
                idx00 = iy0 * W + ix0 + base_off
                for cc, idxv in enumerate((idx00, idx00 + W)):
                    idx_ref[par * NIR + (v * 2 + cc) * 2 + g // 8,
                            pl.ds((g % 8) * 16, 16)] = idxv
                for cc, wv in enumerate(
                        (wy0 * wx0, wy0 * wx1, wy1 * wx0, wy1 * wx1)):
                    wbuf[pl.ds(par * NVIEW * 4 * P + (v * 4 + cc) * P + g * 16,
                               16)] = wv
            for rr in range(4):
                r = v * 4 + rr
                pltpu.async_copy(
                    tab.at[idx_ref.at[par * NIR + r]],
                    rows.at[pl.ds(par * RPC + r * 128, 128)], gsem[par])

    def drain_gathers(par):
        for r in range(NIR):
            pltpu.make_async_copy(
                tab.at[idx_ref.at[par * NIR + r]],
                rows.at[pl.ds(par * RPC + r * 128, 128)], gsem[par]).wait()

    def wait_out(par):
        pltpu.make_async_copy(outb.at[pl.ds(par * P * C, P * C)],
                              out.at[pl.ds(b * NB * C, P * C)],
                              osem[par]).wait()

    def stage_b(k, par):
        start = pbase + k * P

        def group_body(g, c2):
            wvecs = [wbuf[pl.ds(par * NVIEW * 4 * P + i * P + g * 16, 16)]
                     for i in range(NVIEW * 4)]
            for q in range(16):
                p = g * 16 + q
                vsums = []
                for v in range(NVIEW):
                    ra = rows[par * RPC + (v * 2 + 0) * P + p]
                    rb = rows[par * RPC + (v * 2 + 1) * P + p]
                    a0 = lax.bitcast_convert_type(ra << 16, jnp.float32)
                    a1 = lax.bitcast_convert_type(ra & jnp.int32(-65536),
                                                  jnp.float32)
                    b0 = lax.bitcast_convert_type(rb << 16, jnp.float32)
                    b1 = lax.bitcast_convert_type(rb & jnp.int32(-65536),
                                                  jnp.float32)
                    t01 = (a0 * wvecs[v * 4 + 0][q]
                           + a1 * wvecs[v * 4 + 1][q])
                    t23 = (b0 * wvecs[v * 4 + 2][q]
                           + b1 * wvecs[v * 4 + 3][q])
                    vsums.append(t01 + t23)
                acc = ((vsums[0] + vsums[1]) + (vsums[2] + vsums[3])) + vsums[4]
                acc = jnp.clip(acc * (1.0 / NVIEW), 0.0, 1.0)
                outb[pl.ds((par * P + p) * C, C)] = acc
            return c2

        lax.fori_loop(0, P // 16, group_body, 0)
        pltpu.async_copy(outb.at[pl.ds(par * P * C, P * C)],
                         out.at[pl.ds((b * NB + start) * C, P * C)],
                         osem[par])

    fire_coords(0, 0)
    fire_coords(1, 1)
    wait_coords(0)
    stage_a(0)

    def iter_body(i, carry):
        for sub in range(2):
            k = i * 2 + sub
            par, npar = sub, 1 - sub
            fire_coords(lax.rem(k + 2, NCHUNK), par)
            wait_coords(npar)
            stage_a(npar)
            drain_gathers(par)

            @pl.when(i >= 1)
            def _():
                wait_out(par)

            stage_b(k, par)
        return carry

    lax.fori_loop(0, NCHUNK // 2, iter_body, 0)

    wait_coords(1)
    drain_gathers(0)
    wait_out(0)
    wait_out(1)


P2 = 1024
PPW2 = B * NB // NW
NCH2 = PPW2 // P2


@functools.partial(
    pl.kernel,
    out_type=jax.ShapeDtypeStruct((B * NJ * NB,), jnp.float32),
    mesh=_mesh,
    scratch_types=[
        pltpu.VMEM((2 * P2 * C,), jnp.float32),
        pltpu.VMEM((2 * NJ * P2,), jnp.float32),
        pltpu.SemaphoreType.DMA,
        pltpu.SemaphoreType.DMA,
        pltpu.SemaphoreType.DMA,
    ],
    compiler_params=pltpu.CompilerParams(use_tc_tiling_on_sc=False,
                                         needs_layout_passes=False),
)
def _sc_tout(src, dst, ibuf, obuf, isem, osem0, osem1):
    osem = (osem0, osem1)
    wid = lax.axis_index("s") * NC + lax.axis_index("c")
    b = wid // NS
    pbase = (wid % NS) * PPW2
    lane16 = jnp.arange(16, dtype=jnp.int32) * 16

    def fire_in(k, par):
        pltpu.async_copy(
            src.at[pl.ds((b * NB + pbase + k * P2) * C, P2 * C)],
            ibuf.at[pl.ds(par * P2 * C, P2 * C)], isem)

    def wait_in(par):
        pltpu.make_async_copy(src.at[pl.ds(0, P2 * C)],
                              ibuf.at[pl.ds(par * P2 * C, P2 * C)],
                              isem).wait()

    def wait_out(par):
        pltpu.make_async_copy(obuf.at[pl.ds(par * NJ * P2, NJ * P2)],
                              dst.at[pl.ds(0, NJ * P2)], osem[par]).wait()

    def transpose_chunk(k, par):
        def group_body(g, c2):
            gb = par * P2 * C + g * 16 * C
            for c in range(NJ):
                vals = plsc.load_gather(ibuf, [lane16 + (gb + c)])
                obuf[pl.ds(par * NJ * P2 + c * P2 + g * 16, 16)] = vals
            return c2

        lax.fori_loop(0, P2 // 16, group_body, 0)
        start = pbase + k * P2
        for c in range(NJ):
            pltpu.async_copy(
                obuf.at[pl.ds(par * NJ * P2 + c * P2, P2)],
                dst.at[pl.ds((b * NJ + c) * NB + start, P2)],
                osem[par])

    fire_in(0, 0)

    def iter_body(i, carry):
        for sub in range(2):
            k = i * 2 + sub
            par, npar = sub, 1 - sub
            fire_in(lax.rem(k + 1, NCH2), npar)
            wait_in(par)

            @pl.when(i >= 1)
            def _():
                wait_out(par)

            transpose_chunk(k, par)
        return carry

    lax.fori_loop(0, NCH2 // 2, iter_body, 0)
    wait_in(0)
    wait_out(0)
    wait_out(1)


def kernel(heatmaps, sample_grids):
    bb, n, j, h, w = heatmaps.shape
    hm_i = lax.bitcast_convert_type(heatmaps, jnp.int32)
    bf = ((hm_i + 0x7FFF + ((hm_i >> 16) & 1)) >> 16) & 0xFFFF
    nxt = jnp.concatenate([bf[..., 1:], bf[..., -1:]], axis=-1)
    pair_i = bf | (nxt << 16)
    tabi = jnp.transpose(pair_i, (0, 1, 3, 4, 2)).reshape(bb * n * h * w, j)
    tab = jnp.pad(tabi, ((0, 0), (0, C - j)))
    g = sample_grids.reshape(bb * n * NB, 2)
    gx = g[:, 0]
    gy = g[:, 1]
    out = _sc_sample(tab, gx, gy)
    cubes = _sc_tout(out)
    return cubes.reshape(B, NJ, 64, 64, 64)


# speedup vs baseline: 1.0038x; 1.0038x over previous
import functools

import jax
import jax.numpy as jnp
from jax import lax
from jax.experimental import pallas as pl
from jax.experimental.pallas import tpu as pltpu
from jax.experimental.pallas import tpu_sc as plsc

B = 2
NVIEW = 5
NJ = 15
H = 128
W = 240
HW = H * W
NB = 64 * 64 * 64
C = 16

NC = 2
NS = 16
NW = NC * NS

P = 256
NCHUNK = NB // NS // P
PPW = NB // NS
NROW = NVIEW * 2
NIR = NROW * (P // 128)
RPC = NROW * P

_mesh = plsc.VectorSubcoreMesh(
    core_axis_name="c", subcore_axis_name="s", num_cores=NC, num_subcores=NS
)


@functools.partial(
    pl.kernel,
    out_type=jax.ShapeDtypeStruct((B * NB * C,), jnp.float32),
    mesh=_mesh,
    scratch_types=[
        pltpu.VMEM((2 * NIR, 128), jnp.int32),
        pltpu.VMEM((2 * RPC, C), jnp.int32),
        pltpu.VMEM((2 * NVIEW * 4 * P,), jnp.float32),
        pltpu.VMEM((2 * NVIEW * P,), jnp.float32),
        pltpu.VMEM((2 * NVIEW * P,), jnp.float32),
        pltpu.VMEM((2 * P * C,), jnp.float32),
        pltpu.SemaphoreType.DMA,
        pltpu.SemaphoreType.DMA,
        pltpu.SemaphoreType.DMA,
        pltpu.SemaphoreType.DMA,
        pltpu.SemaphoreType.DMA,
    ],
    compiler_params=pltpu.CompilerParams(use_tc_tiling_on_sc=False),
)
def _sc_sample(tab, gx, gy, out, idx_ref, rows, wbuf, cbx, cby, outb,
               csem, gsem0, gsem1, osem0, osem1):
    gsem = (gsem0, gsem1)
    osem = (osem0, osem1)
    wid = lax.axis_index("s") * NC + lax.axis_index("c")
    b = wid // NS
    pbase = (wid % NS) * PPW

    def fire_coords(k, par):
        start = pbase + k * P
        for v in range(NVIEW):
            off = (b * NVIEW + v) * NB + start
            pltpu.async_copy(gx.at[pl.ds(off, P)],
                             cbx.at[pl.ds((par * NVIEW + v) * P, P)], csem)
            pltpu.async_copy(gy.at[pl.ds(off, P)],
                             cby.at[pl.ds((par * NVIEW + v) * P, P)], csem)

    def wait_coords(par):
        pltpu.make_async_copy(gx.at[pl.ds(0, NVIEW * P)],
                              cbx.at[pl.ds(par * NVIEW * P, NVIEW * P)],
                              csem).wait()
        pltpu.make_async_copy(gy.at[pl.ds(0, NVIEW * P)],
                              cby.at[pl.ds(par * NVIEW * P, NVIEW * P)],
                              csem).wait()

    def stage_a(par):
        for v in range(NVIEW):
            base_off = (b * NVIEW + v) * HW
            for g in range(P // 16):
                gxv = cbx[pl.ds((par * NVIEW + v) * P + g * 16, 16)]
                gyv = cby[pl.ds((par * NVIEW + v) * P + g * 16, 16)]
                fx = gxv * (0.5 * (W - 1)) + (0.5 * (W - 1))
                fy = gyv * (0.5 * (H - 1)) + (0.5 * (H - 1))
                ix0 = jnp.maximum(jnp.minimum(fx.astype(jnp.int32), W - 2), 0)
                iy0 = jnp.maximum(jnp.minimum(fy.astype(jnp.int32), H - 2), 0)
                wx1 = fx - ix0.astype(jnp.float32)
                wx0 = 1.0 - wx1
                wy1 = fy - iy0.astype(jnp.float32)
                wy0 = 1.0 - wy1
                idx00 = iy0 * W + ix0 + base_off
                for cc, idxv in enumerate((idx00, idx00 + W)):
                    idx_ref[par * NIR + (v * 2 + cc) * 2 + g // 8,
                            pl.ds((g % 8) * 16, 16)] = idxv
                for cc, wv in enumerate(
                        (wy0 * wx0, wy0 * wx1, wy1 * wx0, wy1 * wx1)):
                    wbuf[pl.ds(par * NVIEW * 4 * P + (v * 4 + cc) * P + g * 16,
                               16)] = wv
            for rr in range(4):
                r = v * 4 + rr
                pltpu.async_copy(
                    tab.at[idx_ref.at[par * NIR + r]],
                    rows.at[pl.ds(par * RPC + r * 128, 128)], gsem[par])

    def drain_gathers(par):
        for r in range(NIR):
            pltpu.make_async_copy(
                tab.at[idx_ref.at[par * NIR + r]],
                rows.at[pl.ds(par * RPC + r * 128, 128)], gsem[par]).wait()

    def wait_out(par):
        pltpu.make_async_copy(outb.at[pl.ds(par * P * C, P * C)],
                              out.at[pl.ds(b * NB * C, P * C)],
                              osem[par]).wait()

    def stage_b(k, par):
        start = pbase + k * P

        def group_body(g, c2):
            wvecs = [wbuf[pl.ds(par * NVIEW * 4 * P + i * P + g * 16, 16)]
                     for i in range(NVIEW * 4)]
            for q in range(16):
                p = g * 16 + q
                vsums = []
                for v in range(NVIEW):
                    ra = rows[par * RPC + (v * 2 + 0) * P + p]
                    rb = rows[par * RPC + (v * 2 + 1) * P + p]
                    a0 = lax.bitcast_convert_type(ra << 16, jnp.float32)
                    a1 = lax.bitcast_convert_type(ra & jnp.int32(-65536),
                                                  jnp.float32)
                    b0 = lax.bitcast_convert_type(rb << 16, jnp.float32)
                    b1 = lax.bitcast_convert_type(rb & jnp.int32(-65536),
                                                  jnp.float32)
                    t01 = (a0 * wvecs[v * 4 + 0][q]
                           + a1 * wvecs[v * 4 + 1][q])
                    t23 = (b0 * wvecs[v * 4 + 2][q]
                           + b1 * wvecs[v * 4 + 3][q])
                    vsums.append(t01 + t23)
                acc = ((vsums[0] + vsums[1]) + (vsums[2] + vsums[3])) + vsums[4]
                acc = jnp.clip(acc * (1.0 / NVIEW), 0.0, 1.0)
                outb[pl.ds((par * P + p) * C, C)] = acc
            return c2

        lax.fori_loop(0, P // 16, group_body, 0)
        pltpu.async_copy(outb.at[pl.ds(par * P * C, P * C)],
                         out.at[pl.ds((b * NB + start) * C, P * C)],
                         osem[par])

    fire_coords(0, 0)
    fire_coords(1, 1)
    wait_coords(0)
    stage_a(0)

    def iter_body(i, carry):
        for sub in range(2):
            k = i * 2 + sub
            par, npar = sub, 1 - sub
            fire_coords(lax.rem(k + 2, NCHUNK), par)
            wait_coords(npar)
            stage_a(npar)
            drain_gathers(par)

            @pl.when(i >= 1)
            def _():
                wait_out(par)

            stage_b(k, par)
        return carry

    lax.fori_loop(0, NCHUNK // 2, iter_body, 0)

    wait_coords(1)
    drain_gathers(0)
    wait_out(0)
    wait_out(1)


P2 = 1024
PPW2 = B * NB // NW
NCH2 = PPW2 // P2


@functools.partial(
    pl.kernel,
    out_type=jax.ShapeDtypeStruct((B * NJ * NB,), jnp.float32),
    mesh=_mesh,
    scratch_types=[
        pltpu.VMEM((2 * P2 * C,), jnp.float32),
        pltpu.VMEM((2 * NJ * P2,), jnp.float32),
        pltpu.SemaphoreType.DMA,
        pltpu.SemaphoreType.DMA,
        pltpu.SemaphoreType.DMA,
    ],
    compiler_params=pltpu.CompilerParams(use_tc_tiling_on_sc=False,
                                         needs_layout_passes=False),
)
def _sc_tout(src, dst, ibuf, obuf, isem, osem0, osem1):
    osem = (osem0, osem1)
    wid = lax.axis_index("s") * NC + lax.axis_index("c")
    b = wid // NS
    pbase = (wid % NS) * PPW2
    lane16 = jnp.arange(16, dtype=jnp.int32) * 16

    def fire_in(k, par):
        pltpu.async_copy(
            src.at[pl.ds((b * NB + pbase + k * P2) * C, P2 * C)],
            ibuf.at[pl.ds(par * P2 * C, P2 * C)], isem)

    def wait_in(par):
        pltpu.make_async_copy(src.at[pl.ds(0, P2 * C)],
                              ibuf.at[pl.ds(par * P2 * C, P2 * C)],
                              isem).wait()

    def wait_out(par):
        pltpu.make_async_copy(obuf.at[pl.ds(par * NJ * P2, NJ * P2)],
                              dst.at[pl.ds(0, NJ * P2)], osem[par]).wait()

    def transpose_chunk(k, par):
        def group_body(g, c2):
            gb = par * P2 * C + g * 16 * C
            for c in range(NJ):
                vals = plsc.load_gather(ibuf, [lane16 + (gb + c)])
                obuf[pl.ds(par * NJ * P2 + c * P2 + g * 16, 16)] = vals
            return c2

        lax.fori_loop(0, P2 // 16, group_body, 0)
        start = pbase + k * P2
        for c in range(NJ):
            pltpu.async_copy(
                obuf.at[pl.ds(par * NJ * P2 + c * P2, P2)],
                dst.at[pl.ds((b * NJ + c) * NB + start, P2)],
                osem[par])

    fire_in(0, 0)

    def iter_body(i, carry):
        for sub in range(2):
            k = i * 2 + sub
            par, npar = sub, 1 - sub
            fire_in(lax.rem(k + 1, NCH2), npar)
            wait_in(par)

            @pl.when(i >= 1)
            def _():
                wait_out(par)

            transpose_chunk(k, par)
        return carry

    lax.fori_loop(0, NCH2 // 2, iter_body, 0)
    wait_in(0)
    wait_out(0)
    wait_out(1)


def kernel(heatmaps, sample_grids):
    bb, n, j, h, w = heatmaps.shape
    hm_i = lax.bitcast_convert_type(heatmaps, jnp.int32)
    bf = ((hm_i + 0x7FFF + ((hm_i >> 16) & 1)) >> 16) & 0xFFFF
    nxt = jnp.concatenate([bf[..., 1:], bf[..., -1:]], axis=-1)
    pair_i = bf | (nxt << 16)
    tabi = jnp.transpose(pair_i, (0, 1, 3, 4, 2)).reshape(bb * n * h * w, j)
    tab = jnp.pad(tabi, ((0, 0), (0, C - j)))
    g = sample_grids.reshape(bb * n * NB, 2)
    gx = g[:, 0]
    gy = g[:, 1]
    out = _sc_sample(tab, gx, gy)
    cubes = _sc_tout(out)
    return cubes.reshape(B, NJ, 64, 64, 64)
